# baseline (speedup 1.0000x reference)
import functools

import jax
import jax.numpy as jnp
from jax import lax
from jax.experimental import pallas as pl
from jax.experimental.pallas import tpu as pltpu
from jax.experimental.pallas import tpu_sc as plsc

VEC_DIM = 64
PADD = 128
BATCH = 16384
CTX = 20
N_NOISE = 10

NC = 2
NS = 16
NW = NC * NS
BPW = BATCH // NW
BPC = BATCH // NC

IDXW = 128
SUB = 128
NQ = 8
CHUNK = NQ * SUB

CTX_IDS_PT = BPW * CTX
TN_IDS_PT = BPW * N_NOISE
N_CTX_CHUNKS = CTX_IDS_PT // CHUNK
N_TN_CHUNKS = TN_IDS_PT // CHUNK
SLOT_ROWS = BATCH * CTX // IDXW
SLOT_RPC = CHUNK // IDXW

_mesh = plsc.VectorSubcoreMesh(core_axis_name="c", subcore_axis_name="s")
_sc_params = pltpu.CompilerParams(use_tc_tiling_on_sc=True)


@functools.partial(
    pl.kernel,
    out_type=[
        jax.ShapeDtypeStruct((BATCH, PADD), jnp.float32),
        jax.ShapeDtypeStruct((BATCH * N_NOISE, PADD), jnp.float32),
    ],
    mesh=_mesh,
    scratch_types=[
        pltpu.VMEM_SHARED((BPC, PADD), jnp.float32),
        pltpu.VMEM((SUB, PADD), jnp.float32),
        pltpu.VMEM((SUB, PADD), jnp.float32),
        pltpu.VMEM((CHUNK,), jnp.int32),
        pltpu.VMEM((SLOT_RPC, IDXW), jnp.int32),
        pltpu.SemaphoreType.DMA,
        pltpu.SemaphoreType.DMA,
        pltpu.SemaphoreType.DMA,
        pltpu.SemaphoreType.DMA,
    ],
    compiler_params=_sc_params,
)
def _sc_words(ctx_h, tn_h, slot_h, word_tbl, ot_tbl, xw_h, or_h,
              xs, rows0, rows1, cidx, slotv, gsem0, gsem1, ssem0, ssem1):
    sid = lax.axis_index("s")
    wid = lax.axis_index("c") * NS + sid

    z = jnp.zeros((16,), jnp.float32)

    @pl.loop(0, SUB)
    def _(i):
        for j in range(PADD // 16):
            rows0[i, pl.ds(j * 16, 16)] = z

    for p in range(BPW // SUB):
        pltpu.sync_copy(rows0, xs.at[pl.ds(sid * BPW + p * SUB, SUB)])

    @pl.loop(0, N_CTX_CHUNKS)
    def _(c):
        base = wid * CTX_IDS_PT + c * CHUNK
        pltpu.sync_copy(ctx_h.at[pl.ds(base, CHUNK)], cidx)
        pltpu.sync_copy(
            slot_h.at[pl.ds(wid * (CTX_IDS_PT // IDXW) + c * SLOT_RPC,
                            SLOT_RPC)],
            slotv,
        )
        bufs = (rows0, rows1)
        gsems = (gsem0, gsem1)
        ssems = (ssem0, ssem1)
        g = [None] * NQ
        s = [None] * NQ
        for q in range(NQ):
            buf, gsem, ssem = bufs[q % 2], gsems[q % 2], ssems[q % 2]
            if q >= 2:
                s[q - 2].wait()
            g[q] = pltpu.async_copy(
                word_tbl.at[cidx.at[pl.ds(q * SUB, SUB)]], buf, gsem
            )
            if q >= 1:
                g[q - 1].wait()
                s[q - 1] = pltpu.async_copy(
                    bufs[(q - 1) % 2],
                    xs.at[slotv.at[q - 1]],
                    ssems[(q - 1) % 2],
                    add=True,
                )
        g[NQ - 1].wait()
        s[NQ - 1] = pltpu.async_copy(
            bufs[(NQ - 1) % 2],
            xs.at[slotv.at[NQ - 1]],
            ssems[(NQ - 1) % 2],
            add=True,
        )
        s[NQ - 2].wait()
        s[NQ - 1].wait()

    pltpu.sync_copy(xs.at[pl.ds(sid * BPW, BPW)], xw_h.at[pl.ds(wid * BPW, BPW)])

    @pl.loop(0, N_TN_CHUNKS)
    def _(c):
        base = wid * TN_IDS_PT + c * CHUNK
        pltpu.sync_copy(tn_h.at[pl.ds(base, CHUNK)], cidx)
        bufs = (rows0, rows1)
        gsems = (gsem0, gsem1)
        ssems = (ssem0, ssem1)
        g = [None] * NQ
        w = [None] * NQ
        for q in range(NQ):
            buf, gsem = bufs[q % 2], gsems[q % 2]
            if q >= 2:
                w[q - 2].wait()
            g[q] = pltpu.async_copy(
                ot_tbl.at[cidx.at[pl.ds(q * SUB, SUB)]], buf, gsem
            )
            if q >= 1:
                g[q - 1].wait()
                w[q - 1] = pltpu.async_copy(
                    bufs[(q - 1) % 2],
                    or_h.at[pl.ds(base + (q - 1) * SUB, SUB)],
                    ssems[(q - 1) % 2],
                )
        g[NQ - 1].wait()
        w[NQ - 1] = pltpu.async_copy(
            bufs[(NQ - 1) % 2],
            or_h.at[pl.ds(base + (NQ - 1) * SUB, SUB)],
            ssems[(NQ - 1) % 2],
        )
        w[NQ - 2].wait()
        w[NQ - 1].wait()


@functools.partial(
    pl.kernel,
    out_type=jax.ShapeDtypeStruct((BATCH, PADD), jnp.float32),
    mesh=_mesh,
    scratch_types=[
        pltpu.VMEM((BPW, PADD), jnp.float32),
        pltpu.VMEM((BPW,), jnp.int32),
        pltpu.SemaphoreType.DMA,
    ],
    compiler_params=_sc_params,
)
def _sc_docs(doc_h, doc_tbl, xd_h, rows, didx, sem):
    wid = lax.axis_index("c") * NS + lax.axis_index("s")
    pltpu.sync_copy(doc_h.at[pl.ds(wid * BPW, BPW)], didx)
    gs = [
        pltpu.async_copy(
            doc_tbl.at[didx.at[pl.ds(k * IDXW, IDXW)]],
            rows.at[pl.ds(k * IDXW, IDXW)],
            sem,
        )
        for k in range(BPW // IDXW)
    ]
    for g in gs:
        g.wait()
    pltpu.sync_copy(rows, xd_h.at[pl.ds(wid * BPW, BPW)])


def _tr_body(x_ref, ot_ref):
    t = x_ref[...].T
    ot_ref[...] = jnp.concatenate([t, jnp.zeros_like(t)], axis=1)


def _tr_pair_body(x_ref, ot_ref):
    t = x_ref[...].T
    t3 = t.reshape(t.shape[0] // 2, 2, VEC_DIM)
    ot_ref[...] = jnp.concatenate([t3[:, 0, :], t3[:, 1, :]], axis=1)


def _tc_transpose_pair(mat, tcol, nblocks):
    npair = nblocks * tcol // 2
    return pl.pallas_call(
        _tr_pair_body,
        grid=(nblocks,),
        in_specs=[pl.BlockSpec((VEC_DIM, tcol), lambda i: (0, i))],
        out_specs=pl.BlockSpec((tcol // 2, PADD), lambda i: (i, 0)),
        out_shape=jax.ShapeDtypeStruct((npair, PADD), jnp.float32),
    )(mat)


def _tc_transpose(mat, tcol, nblocks):
    npad = nblocks * tcol
    return pl.pallas_call(
        _tr_body,
        grid=(nblocks,),
        in_specs=[pl.BlockSpec((VEC_DIM, tcol), lambda i: (0, i))],
        out_specs=pl.BlockSpec((tcol, PADD), lambda i: (i, 0)),
        out_shape=jax.ShapeDtypeStruct((npad, PADD), jnp.float32),
    )(mat)


_BB = 512
_NBLK = BATCH // _BB


def _dot_body(xw_ref, xd_ref, fl_ref, *refs):
    o_refs, out_ref = refs[:N_NOISE], refs[N_NOISE]
    xd = xd_ref[...]
    sel = jnp.where(fl_ref[...] > 0.5, xd[:, VEC_DIM:], xd[:, :VEC_DIM])
    x = xw_ref[...][:, :VEC_DIM] + sel
    cols = [
        jnp.sum(o_refs[n][...][:, :VEC_DIM] * x, axis=-1, keepdims=True)
        for n in range(N_NOISE)
    ]
    out_ref[...] = jnp.concatenate(cols, axis=1)


def _mk_ospec(n):
    return pl.BlockSpec((_BB, PADD), lambda i, n=n: (n * _NBLK + i, 0))


def _tc_dot(xw, xd, flags, orows):
    return pl.pallas_call(
        _dot_body,
        grid=(_NBLK,),
        in_specs=[
            pl.BlockSpec((_BB, PADD), lambda i: (i, 0)),
            pl.BlockSpec((_BB, PADD), lambda i: (i, 0)),
            pl.BlockSpec((_BB, 1), lambda i: (i, 0)),
        ] + [_mk_ospec(n) for n in range(N_NOISE)],
        out_specs=pl.BlockSpec((_BB, N_NOISE), lambda i: (i, 0)),
        out_shape=jax.ShapeDtypeStruct((BATCH, N_NOISE), jnp.float32),
    )(xw, xd, flags, *([orows] * N_NOISE))


def kernel(context_ids, doc_ids, target_noise_ids, Doc, Word, Output):
    ctx_flat = context_ids.astype(jnp.int32).reshape(-1)
    tn_flat = target_noise_ids.astype(jnp.int32).T.reshape(-1)
    doc_flat = doc_ids.astype(jnp.int32)
    slot2d = (
        (jnp.arange(BATCH * CTX, dtype=jnp.int32) // CTX) % BPC
    ).reshape(SLOT_ROWS, IDXW)

    out_t = _tc_transpose(Output, 8192, 13)
    word_t = _tc_transpose(Word.T, 8192, 13)
    doc_t = _tc_transpose_pair(Doc.T, 16384, 62)

    xw, orows = _sc_words(ctx_flat, tn_flat, slot2d, word_t, out_t)
    xd = _sc_docs(doc_flat // 2, doc_t)
    flags = (doc_flat % 2).astype(jnp.float32).reshape(BATCH, 1)
    return _tc_dot(xw, xd, flags, orows)


# speedup vs baseline: 1.4299x; 1.4299x over previous
import functools

import jax
import jax.numpy as jnp
from jax import lax
from jax.experimental import pallas as pl
from jax.experimental.pallas import tpu as pltpu
from jax.experimental.pallas import tpu_sc as plsc

VEC_DIM = 64
PADD = 128
BATCH = 16384
CTX = 20
N_NOISE = 10

NC = 2
NS = 16
NW = NC * NS
BPW = BATCH // NW
BPC = BATCH // NC

IDXW = 128
SUB = 128
NQ = 8
CHUNK = NQ * SUB

CTX_IDS_PT = BPW * CTX
TN_IDS_PT = BPW * N_NOISE
N_CTX_CHUNKS = CTX_IDS_PT // CHUNK
N_TN_CHUNKS = TN_IDS_PT // CHUNK
SLOT_ROWS = BATCH * CTX // IDXW
SLOT_RPC = CHUNK // IDXW

_mesh = plsc.VectorSubcoreMesh(core_axis_name="c", subcore_axis_name="s")
_sc_params = pltpu.CompilerParams(use_tc_tiling_on_sc=True)


@functools.partial(
    pl.kernel,
    out_type=[
        jax.ShapeDtypeStruct((BATCH, PADD), jnp.float32),
        jax.ShapeDtypeStruct((BATCH * N_NOISE, PADD), jnp.float32),
    ],
    mesh=_mesh,
    scratch_types=[
        pltpu.VMEM_SHARED((BPC, PADD), jnp.float32),
        pltpu.VMEM((SUB, PADD), jnp.float32),
        pltpu.VMEM((SUB, PADD), jnp.float32),
        pltpu.VMEM((CHUNK,), jnp.int32),
        pltpu.VMEM((SLOT_RPC, IDXW), jnp.int32),
        pltpu.SemaphoreType.DMA,
        pltpu.SemaphoreType.DMA,
        pltpu.SemaphoreType.DMA,
        pltpu.SemaphoreType.DMA,
    ],
    compiler_params=_sc_params,
)
def _sc_words(ctx_h, tn_h, slot_h, word_tbl, ot_tbl, xw_h, or_h,
              xs, rows0, rows1, cidx, slotv, gsem0, gsem1, ssem0, ssem1):
    sid = lax.axis_index("s")
    wid = lax.axis_index("c") * NS + sid

    z = jnp.zeros((16,), jnp.float32)

    @pl.loop(0, SUB)
    def _(i):
        for j in range(PADD // 16):
            rows0[i, pl.ds(j * 16, 16)] = z

    for p in range(BPW // SUB):
        pltpu.sync_copy(rows0, xs.at[pl.ds(sid * BPW + p * SUB, SUB)])

    @pl.loop(0, N_CTX_CHUNKS)
    def _(c):
        base = wid * CTX_IDS_PT + c * CHUNK
        pltpu.sync_copy(ctx_h.at[pl.ds(base, CHUNK)], cidx)
        pltpu.sync_copy(
            slot_h.at[pl.ds(wid * (CTX_IDS_PT // IDXW) + c * SLOT_RPC,
                            SLOT_RPC)],
            slotv,
        )
        bufs = (rows0, rows1)
        gsems = (gsem0, gsem1)
        ssems = (ssem0, ssem1)
        g = [None] * NQ
        s = [None] * NQ
        for q in range(NQ):
            buf, gsem, ssem = bufs[q % 2], gsems[q % 2], ssems[q % 2]
            if q >= 2:
                s[q - 2].wait()
            g[q] = pltpu.async_copy(
                word_tbl.at[cidx.at[pl.ds(q * SUB, SUB)]], buf, gsem
            )
            if q >= 1:
                g[q - 1].wait()
                s[q - 1] = pltpu.async_copy(
                    bufs[(q - 1) % 2],
                    xs.at[slotv.at[q - 1]],
                    ssems[(q - 1) % 2],
                    add=True,
                )
        g[NQ - 1].wait()
        s[NQ - 1] = pltpu.async_copy(
            bufs[(NQ - 1) % 2],
            xs.at[slotv.at[NQ - 1]],
            ssems[(NQ - 1) % 2],
            add=True,
        )
        s[NQ - 2].wait()
        s[NQ - 1].wait()

    pltpu.sync_copy(xs.at[pl.ds(sid * BPW, BPW)], xw_h.at[pl.ds(wid * BPW, BPW)])

    @pl.loop(0, N_TN_CHUNKS)
    def _(c):
        base = wid * TN_IDS_PT + c * CHUNK
        pltpu.sync_copy(tn_h.at[pl.ds(base, CHUNK)], cidx)
        bufs = (rows0, rows1)
        gsems = (gsem0, gsem1)
        ssems = (ssem0, ssem1)
        g = [None] * NQ
        w = [None] * NQ
        for q in range(NQ):
            buf, gsem = bufs[q % 2], gsems[q % 2]
            if q >= 2:
                w[q - 2].wait()
            g[q] = pltpu.async_copy(
                ot_tbl.at[cidx.at[pl.ds(q * SUB, SUB)]], buf, gsem
            )
            if q >= 1:
                g[q - 1].wait()
                w[q - 1] = pltpu.async_copy(
                    bufs[(q - 1) % 2],
                    or_h.at[pl.ds(base + (q - 1) * SUB, SUB)],
                    ssems[(q - 1) % 2],
                )
        g[NQ - 1].wait()
        w[NQ - 1] = pltpu.async_copy(
            bufs[(NQ - 1) % 2],
            or_h.at[pl.ds(base + (NQ - 1) * SUB, SUB)],
            ssems[(NQ - 1) % 2],
        )
        w[NQ - 2].wait()
        w[NQ - 1].wait()


@functools.partial(
    pl.kernel,
    out_type=jax.ShapeDtypeStruct((BATCH, PADD), jnp.float32),
    mesh=_mesh,
    scratch_types=[
        pltpu.VMEM((BPW, PADD), jnp.float32),
        pltpu.VMEM((BPW,), jnp.int32),
        pltpu.SemaphoreType.DMA,
    ],
    compiler_params=_sc_params,
)
def _sc_docs(doc_h, doc_tbl, xd_h, rows, didx, sem):
    wid = lax.axis_index("c") * NS + lax.axis_index("s")
    pltpu.sync_copy(doc_h.at[pl.ds(wid * BPW, BPW)], didx)
    gs = [
        pltpu.async_copy(
            doc_tbl.at[didx.at[pl.ds(k * IDXW, IDXW)]],
            rows.at[pl.ds(k * IDXW, IDXW)],
            sem,
        )
        for k in range(BPW // IDXW)
    ]
    for g in gs:
        g.wait()
    pltpu.sync_copy(rows, xd_h.at[pl.ds(wid * BPW, BPW)])


def _tr_body(x_ref, ot_ref):
    t = x_ref[...].T
    ot_ref[...] = jnp.concatenate([t, jnp.zeros_like(t)], axis=1)


def _tc_transpose(mat, tcol, nblocks):
    npad = nblocks * tcol
    return pl.pallas_call(
        _tr_body,
        grid=(nblocks,),
        in_specs=[pl.BlockSpec((VEC_DIM, tcol), lambda i: (0, i))],
        out_specs=pl.BlockSpec((tcol, PADD), lambda i: (i, 0)),
        out_shape=jax.ShapeDtypeStruct((npad, PADD), jnp.float32),
    )(mat)


_BB = 512
_NBLK = BATCH // _BB


def _dot_body(xw_ref, xd_ref, *refs):
    o_refs, out_ref = refs[:N_NOISE], refs[N_NOISE]
    x = xw_ref[...][:, :VEC_DIM] + xd_ref[...][:, :VEC_DIM]
    cols = [
        jnp.sum(o_refs[n][...][:, :VEC_DIM] * x, axis=-1, keepdims=True)
        for n in range(N_NOISE)
    ]
    out_ref[...] = jnp.concatenate(cols, axis=1)


def _mk_ospec(n):
    return pl.BlockSpec((_BB, PADD), lambda i, n=n: (n * _NBLK + i, 0))


def _tc_dot(xw, xd, orows):
    return pl.pallas_call(
        _dot_body,
        grid=(_NBLK,),
        in_specs=[
            pl.BlockSpec((_BB, PADD), lambda i: (i, 0)),
            pl.BlockSpec((_BB, PADD), lambda i: (i, 0)),
        ] + [_mk_ospec(n) for n in range(N_NOISE)],
        out_specs=pl.BlockSpec((_BB, N_NOISE), lambda i: (i, 0)),
        out_shape=jax.ShapeDtypeStruct((BATCH, N_NOISE), jnp.float32),
    )(xw, xd, *([orows] * N_NOISE))


def kernel(context_ids, doc_ids, target_noise_ids, Doc, Word, Output):
    ctx_flat = context_ids.astype(jnp.int32).reshape(-1)
    tn_flat = target_noise_ids.astype(jnp.int32).T.reshape(-1)
    doc_flat = doc_ids.astype(jnp.int32)
    slot2d = (
        (jnp.arange(BATCH * CTX, dtype=jnp.int32) // CTX) % BPC
    ).reshape(SLOT_ROWS, IDXW)

    out_t = _tc_transpose(Output, 8192, 13)
    word_t = _tc_transpose(Word.T, 8192, 13)
    doc_t = _tc_transpose(Doc.T, 32768, 31)

    xw, orows = _sc_words(ctx_flat, tn_flat, slot2d, word_t, out_t)
    xd = _sc_docs(doc_flat, doc_t)
    return _tc_dot(xw, xd, orows)
